# baseline (speedup 1.0000x reference)
import functools

import jax
import jax.numpy as jnp
from jax import lax
from jax.experimental import pallas as pl
from jax.experimental.pallas import tpu as pltpu
from jax.experimental.pallas import tpu_sc as plsc

B = 4
N = 4096
K = 16
EPS = 1e-12
NSUB = 32
SUBS_PER_BATCH = NSUB // B
NCHUNK = N // 16

R_SC = 3072
ROWS_PER_SUB = R_SC // SUBS_PER_BATCH
TCR = 256
NB_TC = (N - R_SC) // TCR


def _sqrt16(x):
    i = lax.bitcast_convert_type(x, jnp.int32)
    y = lax.bitcast_convert_type((i >> 1) + jnp.int32(0x1FBD1DF5), jnp.float32)
    for _ in range(3):
        y = jnp.float32(0.5) * (y + x / y)
    return y


def _sc_body(pts, out, xs, ys, zs, d2b0, d2b1, sv0, sv1, accv):
    cid = lax.axis_index("c")
    sid = lax.axis_index("s")
    wid = cid * 16 + sid
    b = wid // SUBS_PER_BATCH
    q0 = (wid % SUBS_PER_BATCH) * ROWS_PER_SUB

    pltpu.sync_copy(pts.at[b * 3 + 0], xs)
    pltpu.sync_copy(pts.at[b * 3 + 1], ys)
    pltpu.sync_copy(pts.at[b * 3 + 2], zs)

    inf = jnp.float32(jnp.inf)
    iot = lax.iota(jnp.int32, 16)
    inf16 = jnp.full((16,), inf, jnp.float32)

    def select16(surv, n):
        best = lax.sort(surv[pl.ds(1, 16)])
        nch = (n - 16 + 15) // 16

        def p3(j, bst):
            base = 17 + j * 16
            v = surv[pl.ds(base, 16)]
            v = jnp.where(base - 1 + iot < n, v, inf)
            vs = lax.sort(v)
            return lax.sort(jnp.minimum(bst, lax.rev(vs, (0,))))

        best = lax.fori_loop(0, nch, p3, best)
        return _sqrt16(best + jnp.float32(EPS))

    def pair_step(p, acc):
        qa = jnp.full((16,), q0 + 2 * p, jnp.int32)
        qb = qa + 1
        qx0 = plsc.load_gather(xs, [qa])
        qy0 = plsc.load_gather(ys, [qa])
        qz0 = plsc.load_gather(zs, [qa])
        qx1 = plsc.load_gather(xs, [qb])
        qy1 = plsc.load_gather(ys, [qb])
        qz1 = plsc.load_gather(zs, [qb])

        @plsc.parallel_loop(0, NCHUNK // 2, carry=(inf16, inf16, inf16, inf16),
                            unroll=2)
        def p1(c, ms):
            m0a, m0b, m1a, m1b = ms
            sa = pl.ds(c * 32, 16)
            sb = pl.ds(c * 32 + 16, 16)
            cxa = xs[sa]
            cya = ys[sa]
            cza = zs[sa]
            cxb = xs[sb]
            cyb = ys[sb]
            czb = zs[sb]
            dx0a = cxa - qx0
            dy0a = cya - qy0
            dz0a = cza - qz0
            dx1a = cxa - qx1
            dy1a = cya - qy1
            dz1a = cza - qz1
            dx0b = cxb - qx0
            dy0b = cyb - qy0
            dz0b = czb - qz0
            dx1b = cxb - qx1
            dy1b = cyb - qy1
            dz1b = czb - qz1
            d20a = dx0a * dx0a + dy0a * dy0a + dz0a * dz0a
            d21a = dx1a * dx1a + dy1a * dy1a + dz1a * dz1a
            d20b = dx0b * dx0b + dy0b * dy0b + dz0b * dz0b
            d21b = dx1b * dx1b + dy1b * dy1b + dz1b * dz1b
            d2b0[sa] = d20a
            d2b0[sb] = d20b
            d2b1[sa] = d21a
            d2b1[sb] = d21b
            return (jnp.minimum(m0a, d20a), jnp.minimum(m0b, d20b),
                    jnp.minimum(m1a, d21a), jnp.minimum(m1b, d21b))

        tau0 = jnp.max(jnp.minimum(p1[0], p1[1]))
        tau1 = jnp.max(jnp.minimum(p1[2], p1[3]))

        zero16 = jnp.zeros((16,), jnp.int32)

        one16 = jnp.ones((16,), jnp.int32)

        @plsc.parallel_loop(0, NCHUNK, carry=(zero16, zero16), unroll=4)
        def p2(c, offs):
            off0, off1 = offs
            sl = pl.ds(c * 16, 16)
            v0 = d2b0[sl]
            v1 = d2b1[sl]
            k0 = v0 <= tau0
            k1 = v1 <= tau1
            pos0 = plsc.cumsum(one16, mask=k0) + off0
            pos1 = plsc.cumsum(one16, mask=k1) + off1
            plsc.store_scatter(sv0, [pos0], v0, mask=k0)
            plsc.store_scatter(sv1, [pos1], v1, mask=k1)
            return (off0 + plsc.all_reduce_population_count(k0),
                    off1 + plsc.all_reduce_population_count(k1))

        n0 = jnp.max(p2[0])
        n1 = jnp.max(p2[1])
        acc = acc + select16(sv0, n0)
        acc = acc + select16(sv1, n1)
        return acc

    acc = lax.fori_loop(0, ROWS_PER_SUB // 2, pair_step,
                        jnp.zeros((16,), jnp.float32))
    accv[...] = acc
    pltpu.sync_copy(accv, out.at[wid])


def _sc_rows(pts):
    sc_call = pl.kernel(
        _sc_body,
        out_type=jax.ShapeDtypeStruct((NSUB, 16), jnp.float32),
        mesh=plsc.VectorSubcoreMesh(core_axis_name="c", subcore_axis_name="s"),
        compiler_params=pltpu.CompilerParams(needs_layout_passes=False),
        scratch_types=[
            pltpu.VMEM((N,), jnp.float32),
            pltpu.VMEM((N,), jnp.float32),
            pltpu.VMEM((N,), jnp.float32),
            pltpu.VMEM((N,), jnp.float32),
            pltpu.VMEM((N,), jnp.float32),
            pltpu.VMEM((N + 32,), jnp.float32),
            pltpu.VMEM((N + 32,), jnp.float32),
            pltpu.VMEM((16,), jnp.float32),
        ],
    )
    return sc_call(pts)


def _tc_rows(pc, pts_t):
    def body(q_ref, c_ref, o_ref):
        q = q_ref[0]
        qx = q[:, 0:1]
        qy = q[:, 1:2]
        qz = q[:, 2:3]
        cx = c_ref[0, 0:1, :]
        cy = c_ref[0, 1:2, :]
        cz = c_ref[0, 2:3, :]
        dx = qx - cx
        dy = qy - cy
        dz = qz - cz
        d2 = dx * dx + dy * dy + dz * dz
        total = jnp.zeros((TCR, 1), jnp.float32)
        rem = jnp.full((TCR, 1), float(K), jnp.float32)
        inf = jnp.float32(jnp.inf)
        for _ in range(K):
            mn = jnp.min(d2, axis=1, keepdims=True)
            eq = d2 == mn
            cnt = jnp.sum(jnp.where(eq, 1.0, 0.0), axis=1, keepdims=True)
            take = jnp.minimum(cnt, rem)
            total += take * jnp.sqrt(mn + jnp.float32(EPS))
            rem = rem - take
            d2 = jnp.where(eq, inf, d2)
        o_ref[...] = jnp.broadcast_to(jnp.sum(total), (1, 1, 1, 1))

    return pl.pallas_call(
        body,
        grid=(B, NB_TC),
        in_specs=[
            pl.BlockSpec((1, TCR, 3), lambda b, j: (b, (R_SC // TCR) + j, 0)),
            pl.BlockSpec((1, 3, N), lambda b, j: (b, 0, 0)),
        ],
        out_specs=pl.BlockSpec((1, 1, 1, 1), lambda b, j: (b, j, 0, 0)),
        out_shape=jax.ShapeDtypeStruct((B, NB_TC, 1, 1), jnp.float32),
    )(pc, pts_t)


def _tc_reduce(parts_sc, parts_tc):
    def body(a_ref, b_ref, o_ref):
        val = (jnp.sum(a_ref[...]) + jnp.sum(b_ref[...])) * jnp.float32(
            1.0 / (B * N))
        o_ref[...] = jnp.broadcast_to(val, (1, 1))

    return pl.pallas_call(
        body,
        out_shape=jax.ShapeDtypeStruct((1, 1), jnp.float32),
    )(parts_sc, parts_tc)


@jax.jit
def kernel(point_cloud):
    pts_t = jnp.transpose(point_cloud, (0, 2, 1))
    pts = pts_t.reshape(B * 3, N)
    parts_sc = _sc_rows(pts)
    parts_tc = _tc_rows(point_cloud, pts_t).reshape(B, NB_TC)
    return _tc_reduce(parts_sc, parts_tc).reshape(())


# speedup vs baseline: 1.0402x; 1.0402x over previous
import functools

import jax
import jax.numpy as jnp
from jax import lax
from jax.experimental import pallas as pl
from jax.experimental.pallas import tpu as pltpu
from jax.experimental.pallas import tpu_sc as plsc

B = 4
N = 4096
K = 16
EPS = 1e-12
NSUB = 32
SUBS_PER_BATCH = NSUB // B
NCHUNK = N // 16

R_SC = 2944
ROWS_PER_SUB = R_SC // SUBS_PER_BATCH
TCR = 128
NB_TC = (N - R_SC) // TCR


def _sqrt16(x):
    i = lax.bitcast_convert_type(x, jnp.int32)
    y = lax.bitcast_convert_type((i >> 1) + jnp.int32(0x1FBD1DF5), jnp.float32)
    for _ in range(3):
        y = jnp.float32(0.5) * (y + x / y)
    return y


def _sc_body(pts, out, xs, ys, zs, d2b0, d2b1, sv0, sv1, accv):
    cid = lax.axis_index("c")
    sid = lax.axis_index("s")
    wid = cid * 16 + sid
    b = wid // SUBS_PER_BATCH
    q0 = (wid % SUBS_PER_BATCH) * ROWS_PER_SUB

    pltpu.sync_copy(pts.at[b * 3 + 0], xs)
    pltpu.sync_copy(pts.at[b * 3 + 1], ys)
    pltpu.sync_copy(pts.at[b * 3 + 2], zs)

    inf = jnp.float32(jnp.inf)
    iot = lax.iota(jnp.int32, 16)
    inf16 = jnp.full((16,), inf, jnp.float32)

    def select16(surv, n):
        best = lax.sort(surv[pl.ds(1, 16)])
        nch = (n - 16 + 15) // 16

        def p3(j, bst):
            base = 17 + j * 16
            v = surv[pl.ds(base, 16)]
            v = jnp.where(base - 1 + iot < n, v, inf)
            vs = lax.sort(v)
            return lax.sort(jnp.minimum(bst, lax.rev(vs, (0,))))

        best = lax.fori_loop(0, nch, p3, best)
        return _sqrt16(best + jnp.float32(EPS))

    def pair_step(p, acc):
        qa = jnp.full((16,), q0 + 2 * p, jnp.int32)
        qb = qa + 1
        qx0 = plsc.load_gather(xs, [qa])
        qy0 = plsc.load_gather(ys, [qa])
        qz0 = plsc.load_gather(zs, [qa])
        qx1 = plsc.load_gather(xs, [qb])
        qy1 = plsc.load_gather(ys, [qb])
        qz1 = plsc.load_gather(zs, [qb])

        @plsc.parallel_loop(0, NCHUNK // 2, carry=(inf16, inf16, inf16, inf16),
                            unroll=2)
        def p1(c, ms):
            m0a, m0b, m1a, m1b = ms
            sa = pl.ds(c * 32, 16)
            sb = pl.ds(c * 32 + 16, 16)
            cxa = xs[sa]
            cya = ys[sa]
            cza = zs[sa]
            cxb = xs[sb]
            cyb = ys[sb]
            czb = zs[sb]
            dx0a = cxa - qx0
            dy0a = cya - qy0
            dz0a = cza - qz0
            dx1a = cxa - qx1
            dy1a = cya - qy1
            dz1a = cza - qz1
            dx0b = cxb - qx0
            dy0b = cyb - qy0
            dz0b = czb - qz0
            dx1b = cxb - qx1
            dy1b = cyb - qy1
            dz1b = czb - qz1
            d20a = dx0a * dx0a + dy0a * dy0a + dz0a * dz0a
            d21a = dx1a * dx1a + dy1a * dy1a + dz1a * dz1a
            d20b = dx0b * dx0b + dy0b * dy0b + dz0b * dz0b
            d21b = dx1b * dx1b + dy1b * dy1b + dz1b * dz1b
            d2b0[sa] = d20a
            d2b0[sb] = d20b
            d2b1[sa] = d21a
            d2b1[sb] = d21b
            return (jnp.minimum(m0a, d20a), jnp.minimum(m0b, d20b),
                    jnp.minimum(m1a, d21a), jnp.minimum(m1b, d21b))

        tau0 = jnp.max(jnp.minimum(p1[0], p1[1]))
        tau1 = jnp.max(jnp.minimum(p1[2], p1[3]))

        zero16 = jnp.zeros((16,), jnp.int32)

        one16 = jnp.ones((16,), jnp.int32)

        @plsc.parallel_loop(0, NCHUNK, carry=(zero16, zero16), unroll=4)
        def p2(c, offs):
            off0, off1 = offs
            sl = pl.ds(c * 16, 16)
            v0 = d2b0[sl]
            v1 = d2b1[sl]
            k0 = v0 <= tau0
            k1 = v1 <= tau1
            pos0 = plsc.cumsum(one16, mask=k0) + off0
            pos1 = plsc.cumsum(one16, mask=k1) + off1
            plsc.store_scatter(sv0, [pos0], v0, mask=k0)
            plsc.store_scatter(sv1, [pos1], v1, mask=k1)
            return (off0 + plsc.all_reduce_population_count(k0),
                    off1 + plsc.all_reduce_population_count(k1))

        n0 = jnp.max(p2[0])
        n1 = jnp.max(p2[1])
        acc = acc + select16(sv0, n0)
        acc = acc + select16(sv1, n1)
        return acc

    acc = lax.fori_loop(0, ROWS_PER_SUB // 2, pair_step,
                        jnp.zeros((16,), jnp.float32))
    accv[...] = acc
    pltpu.sync_copy(accv, out.at[wid])


def _sc_rows(pts):
    sc_call = pl.kernel(
        _sc_body,
        out_type=jax.ShapeDtypeStruct((NSUB, 16), jnp.float32),
        mesh=plsc.VectorSubcoreMesh(core_axis_name="c", subcore_axis_name="s"),
        compiler_params=pltpu.CompilerParams(needs_layout_passes=False),
        scratch_types=[
            pltpu.VMEM((N,), jnp.float32),
            pltpu.VMEM((N,), jnp.float32),
            pltpu.VMEM((N,), jnp.float32),
            pltpu.VMEM((N,), jnp.float32),
            pltpu.VMEM((N,), jnp.float32),
            pltpu.VMEM((N + 32,), jnp.float32),
            pltpu.VMEM((N + 32,), jnp.float32),
            pltpu.VMEM((16,), jnp.float32),
        ],
    )
    return sc_call(pts)


def _tc_rows(pc, pts_t):
    def body(q_ref, c_ref, o_ref):
        q = q_ref[0]
        qx = q[:, 0:1]
        qy = q[:, 1:2]
        qz = q[:, 2:3]
        cx = c_ref[0, 0:1, :]
        cy = c_ref[0, 1:2, :]
        cz = c_ref[0, 2:3, :]
        dx = qx - cx
        dy = qy - cy
        dz = qz - cz
        d2 = dx * dx + dy * dy + dz * dz
        total = jnp.zeros((TCR, 1), jnp.float32)
        rem = jnp.full((TCR, 1), float(K), jnp.float32)
        inf = jnp.float32(jnp.inf)
        for _ in range(K):
            mn = jnp.min(d2, axis=1, keepdims=True)
            eq = d2 == mn
            cnt = jnp.sum(jnp.where(eq, 1.0, 0.0), axis=1, keepdims=True)
            take = jnp.minimum(cnt, rem)
            total += take * jnp.sqrt(mn + jnp.float32(EPS))
            rem = rem - take
            d2 = jnp.where(eq, inf, d2)
        o_ref[...] = jnp.broadcast_to(jnp.sum(total), (1, 1, 1, 1))

    return pl.pallas_call(
        body,
        grid=(B, NB_TC),
        in_specs=[
            pl.BlockSpec((1, TCR, 3), lambda b, j: (b, (R_SC // TCR) + j, 0)),
            pl.BlockSpec((1, 3, N), lambda b, j: (b, 0, 0)),
        ],
        out_specs=pl.BlockSpec((1, 1, 1, 1), lambda b, j: (b, j, 0, 0)),
        out_shape=jax.ShapeDtypeStruct((B, NB_TC, 1, 1), jnp.float32),
    )(pc, pts_t)


def _tc_reduce(parts_sc, parts_tc):
    def body(a_ref, b_ref, o_ref):
        val = (jnp.sum(a_ref[...]) + jnp.sum(b_ref[...])) * jnp.float32(
            1.0 / (B * N))
        o_ref[...] = jnp.broadcast_to(val, (1, 1))

    return pl.pallas_call(
        body,
        out_shape=jax.ShapeDtypeStruct((1, 1), jnp.float32),
    )(parts_sc, parts_tc)


@jax.jit
def kernel(point_cloud):
    pts_t = jnp.transpose(point_cloud, (0, 2, 1))
    pts = pts_t.reshape(B * 3, N)
    parts_sc = _sc_rows(pts)
    parts_tc = _tc_rows(point_cloud, pts_t).reshape(B, NB_TC)
    return _tc_reduce(parts_sc, parts_tc).reshape(())
